# baseline (device time: 101472 ns/iter reference)
import jax
import jax.numpy as jnp
from jax import lax
from jax.experimental import pallas as pl
from jax.experimental.pallas import tpu as pltpu

N_DEV = 32
LOG_N = 5


def _allreduce_sum(partial):
    m, n = partial.shape

    def body(in_ref, out_ref, acc_ref, recv_ref, send_sems, recv_sems):
        my = lax.axis_index("i")

        barrier = pltpu.get_barrier_semaphore()
        for s in range(LOG_N):
            pl.semaphore_signal(
                barrier,
                inc=1,
                device_id=(my ^ (1 << s),),
                device_id_type=pl.DeviceIdType.MESH,
            )
        pl.semaphore_wait(barrier, LOG_N)

        acc_ref[...] = in_ref[...]
        for s in range(LOG_N):
            partner = my ^ (1 << s)
            rdma = pltpu.make_async_remote_copy(
                src_ref=acc_ref,
                dst_ref=recv_ref.at[s],
                send_sem=send_sems.at[s],
                recv_sem=recv_sems.at[s],
                device_id=(partner,),
                device_id_type=pl.DeviceIdType.MESH,
            )
            rdma.start()
            rdma.wait()
            acc_ref[...] = acc_ref[...] + recv_ref[s]
        out_ref[...] = acc_ref[...]

    return pl.pallas_call(
        body,
        out_shape=jax.ShapeDtypeStruct((m, n), partial.dtype),
        in_specs=[pl.BlockSpec(memory_space=pltpu.VMEM)],
        out_specs=pl.BlockSpec(memory_space=pltpu.VMEM),
        scratch_shapes=[
            pltpu.VMEM((m, n), partial.dtype),
            pltpu.VMEM((LOG_N, m, n), partial.dtype),
            pltpu.SemaphoreType.DMA((LOG_N,)),
            pltpu.SemaphoreType.DMA((LOG_N,)),
        ],
        compiler_params=pltpu.CompilerParams(collective_id=0),
    )(partial)


def kernel(x, Wq, K_ext, V_ext, Wo):
    B, Sq, d_model = x.shape
    _, Skv, h_per, Dh = K_ext.shape
    cols = h_per * Dh

    my = lax.axis_index("i")
    Wq_l = lax.dynamic_slice(Wq, (0, my * cols), (d_model, cols))
    Wo_l = lax.dynamic_slice(Wo, (my * cols, 0), (cols, Wo.shape[1]))

    xb = x.astype(jnp.bfloat16)
    Q = jnp.einsum(
        "bsd,dc->bsc", xb, Wq_l.astype(jnp.bfloat16),
        preferred_element_type=jnp.bfloat16,
    ).reshape(B, Sq, h_per, Dh)

    K = K_ext.astype(jnp.bfloat16)
    V = V_ext.astype(jnp.bfloat16)
    scores = jnp.einsum(
        "bihd,bjhd->bhij", Q, K, preferred_element_type=jnp.float32
    ) * 0.125

    qb = (jnp.arange(Sq) // 64)[:, None]
    kb = (jnp.arange(Skv) // 64)[None, :]
    mask = (qb == kb) | (kb == 0) | ((qb + kb) % 3 == 0)
    scores = jnp.where(mask[None, None], scores, -1e9)
    smax = scores.max(axis=-1, keepdims=True)
    w = jnp.exp(scores - smax)
    row_keep = mask.any(axis=1)
    w_sum = jnp.where(
        row_keep[None, None, :, None], w.sum(axis=-1, keepdims=True), 1.0
    )
    w = jnp.where(row_keep[None, None, :, None], w / w_sum, 0.0)

    ctx = jnp.einsum(
        "bhij,bjhd->bihd", w.astype(jnp.bfloat16), V,
        preferred_element_type=jnp.bfloat16,
    ).reshape(B, Sq, cols)
    part = jnp.einsum(
        "bsc,cd->bsd", ctx, Wo_l.astype(jnp.bfloat16),
        preferred_element_type=jnp.float32,
    )

    out = _allreduce_sum(part.reshape(B * Sq, d_model))
    return out.reshape(B, Sq, d_model)


# device time: 62041 ns/iter; 1.6356x vs baseline; 1.6356x over previous
import jax
import jax.numpy as jnp
from jax import lax
from jax.experimental import pallas as pl
from jax.experimental.pallas import tpu as pltpu

N_DEV = 32
LOG_N = 5


def _allreduce_sum(partial):
    m, n = partial.shape

    def body(in_ref, out_ref, acc_ref, send_ref, recv_ref, send_sems, recv_sems):
        my = lax.axis_index("i")

        barrier = pltpu.get_barrier_semaphore()
        for s in range(LOG_N):
            pl.semaphore_signal(
                barrier,
                inc=1,
                device_id=(my ^ (1 << s),),
                device_id_type=pl.DeviceIdType.MESH,
            )
        pl.semaphore_wait(barrier, LOG_N)

        acc_ref[...] = in_ref[...]
        for s in range(LOG_N):
            partner = my ^ (1 << s)
            send_ref[...] = acc_ref[...].astype(jnp.bfloat16)
            rdma = pltpu.make_async_remote_copy(
                src_ref=send_ref,
                dst_ref=recv_ref.at[s],
                send_sem=send_sems.at[s],
                recv_sem=recv_sems.at[s],
                device_id=(partner,),
                device_id_type=pl.DeviceIdType.MESH,
            )
            rdma.start()
            rdma.wait()
            acc_ref[...] = acc_ref[...] + recv_ref[s].astype(jnp.float32)
        out_ref[...] = acc_ref[...]

    return pl.pallas_call(
        body,
        out_shape=jax.ShapeDtypeStruct((m, n), partial.dtype),
        in_specs=[pl.BlockSpec(memory_space=pltpu.VMEM)],
        out_specs=pl.BlockSpec(memory_space=pltpu.VMEM),
        scratch_shapes=[
            pltpu.VMEM((m, n), partial.dtype),
            pltpu.VMEM((m, n), jnp.bfloat16),
            pltpu.VMEM((LOG_N, m, n), jnp.bfloat16),
            pltpu.SemaphoreType.DMA((LOG_N,)),
            pltpu.SemaphoreType.DMA((LOG_N,)),
        ],
        compiler_params=pltpu.CompilerParams(collective_id=0),
    )(partial)


def kernel(x, Wq, K_ext, V_ext, Wo):
    B, Sq, d_model = x.shape
    _, Skv, h_per, Dh = K_ext.shape
    cols = h_per * Dh

    my = lax.axis_index("i")
    Wq_l = lax.dynamic_slice(Wq, (0, my * cols), (d_model, cols))
    Wo_l = lax.dynamic_slice(Wo, (my * cols, 0), (cols, Wo.shape[1]))

    xb = x.astype(jnp.bfloat16)
    Q = jnp.einsum(
        "bsd,dc->bsc", xb, Wq_l.astype(jnp.bfloat16),
        preferred_element_type=jnp.bfloat16,
    ).reshape(B, Sq, h_per, Dh)

    K = K_ext.astype(jnp.bfloat16)
    V = V_ext.astype(jnp.bfloat16)
    scores = jnp.einsum(
        "bihd,bjhd->bhij", Q, K, preferred_element_type=jnp.float32
    ) * 0.125

    qb = (jnp.arange(Sq) // 64)[:, None]
    kb = (jnp.arange(Skv) // 64)[None, :]
    mask = (qb == kb) | (kb == 0) | ((qb + kb) % 3 == 0)
    scores = jnp.where(mask[None, None], scores, -1e9)
    smax = scores.max(axis=-1, keepdims=True)
    w = jnp.exp(scores - smax)
    row_keep = mask.any(axis=1)
    w_sum = jnp.where(
        row_keep[None, None, :, None], w.sum(axis=-1, keepdims=True), 1.0
    )
    w = jnp.where(row_keep[None, None, :, None], w / w_sum, 0.0)

    ctx = jnp.einsum(
        "bhij,bjhd->bihd", w.astype(jnp.bfloat16), V,
        preferred_element_type=jnp.bfloat16,
    ).reshape(B, Sq, cols)
    part = jnp.einsum(
        "bsc,cd->bsd", ctx, Wo_l.astype(jnp.bfloat16),
        preferred_element_type=jnp.float32,
    )

    out = _allreduce_sum(part.reshape(B * Sq, d_model))
    return out.reshape(B, Sq, d_model)


# device time: 38556 ns/iter; 2.6318x vs baseline; 1.6091x over previous
import jax
import jax.numpy as jnp
from jax import lax
from jax.experimental import pallas as pl
from jax.experimental.pallas import tpu as pltpu

N_DEV = 32
LOG_N = 5


def _rev5(c: int) -> int:
    return int(format(c, "05b")[::-1], 2)


def _rev5_traced(v):
    return ((v & 1) << 4) | ((v & 2) << 2) | (v & 4) | ((v >> 2) & 2) | ((v >> 4) & 1)


def _allreduce_sum(partial):
    m, n = partial.shape
    rows = m // N_DEV

    def body(in_ref, out_ref, acc_ref, ag_ref, *rest):
        rs_send = rest[0:LOG_N]
        rs_recv = rest[LOG_N : 2 * LOG_N]
        rs_send_sems, rs_recv_sems, ag_send_sems, ag_recv_sems = rest[2 * LOG_N :]

        my = lax.axis_index("i")
        rev_my = _rev5_traced(my)

        barrier = pltpu.get_barrier_semaphore()
        for s in range(LOG_N):
            pl.semaphore_signal(
                barrier,
                inc=1,
                device_id=(my ^ (1 << s),),
                device_id_type=pl.DeviceIdType.MESH,
            )
        pl.semaphore_wait(barrier, LOG_N)

        for j in range(N_DEV):
            acc_ref[j * rows : (j + 1) * rows, :] = in_ref[
                _rev5(j) * rows : (_rev5(j) + 1) * rows, :
            ]

        for s in range(LOG_N):
            partner = my ^ (1 << s)
            rev_p = _rev5_traced(partner)
            shift = LOG_N - 1 - s
            nrows = (1 << shift) * rows
            send_start = ((rev_p >> shift) << shift) * rows
            keep_start = ((rev_my >> shift) << shift) * rows

            rs_send[s][...] = acc_ref[pl.ds(send_start, nrows), :].astype(
                jnp.bfloat16
            )
            rdma = pltpu.make_async_remote_copy(
                src_ref=rs_send[s],
                dst_ref=rs_recv[s],
                send_sem=rs_send_sems,
                recv_sem=rs_recv_sems.at[s],
                device_id=(partner,),
                device_id_type=pl.DeviceIdType.MESH,
            )
            rdma.start()
            rdma.wait()
            acc_ref[pl.ds(keep_start, nrows), :] = (
                acc_ref[pl.ds(keep_start, nrows), :]
                + rs_recv[s][...].astype(jnp.float32)
            )

        ag_ref[pl.ds(rev_my * rows, rows), :] = acc_ref[
            pl.ds(rev_my * rows, rows), :
        ].astype(jnp.bfloat16)

        for s in range(LOG_N):
            partner = my ^ (1 << (LOG_N - 1 - s))
            nrows = (1 << s) * rows
            my_start = ((rev_my >> s) << s) * rows
            rdma = pltpu.make_async_remote_copy(
                src_ref=ag_ref.at[pl.ds(my_start, nrows), :],
                dst_ref=ag_ref.at[pl.ds(my_start, nrows), :],
                send_sem=ag_send_sems,
                recv_sem=ag_recv_sems.at[s],
                device_id=(partner,),
                device_id_type=pl.DeviceIdType.MESH,
            )
            rdma.start()
            rdma.wait()

        for c in range(N_DEV):
            out_ref[c * rows : (c + 1) * rows, :] = ag_ref[
                _rev5(c) * rows : (_rev5(c) + 1) * rows, :
            ].astype(jnp.float32)

    return pl.pallas_call(
        body,
        out_shape=jax.ShapeDtypeStruct((m, n), partial.dtype),
        in_specs=[pl.BlockSpec(memory_space=pltpu.VMEM)],
        out_specs=pl.BlockSpec(memory_space=pltpu.VMEM),
        scratch_shapes=[
            pltpu.VMEM((m, n), partial.dtype),
            pltpu.VMEM((m, n), jnp.bfloat16),
            *[
                pltpu.VMEM(((m // 2) >> s, n), jnp.bfloat16)
                for s in range(LOG_N)
            ],
            *[
                pltpu.VMEM(((m // 2) >> s, n), jnp.bfloat16)
                for s in range(LOG_N)
            ],
            pltpu.SemaphoreType.DMA,
            pltpu.SemaphoreType.DMA((LOG_N,)),
            pltpu.SemaphoreType.DMA,
            pltpu.SemaphoreType.DMA((LOG_N,)),
        ],
        compiler_params=pltpu.CompilerParams(collective_id=0),
    )(partial)


def kernel(x, Wq, K_ext, V_ext, Wo):
    B, Sq, d_model = x.shape
    _, Skv, h_per, Dh = K_ext.shape
    cols = h_per * Dh

    my = lax.axis_index("i")
    Wq_l = lax.dynamic_slice(Wq, (0, my * cols), (d_model, cols))
    Wo_l = lax.dynamic_slice(Wo, (my * cols, 0), (cols, Wo.shape[1]))

    xb = x.astype(jnp.bfloat16)
    Q = jnp.einsum(
        "bsd,dc->bsc", xb, Wq_l.astype(jnp.bfloat16),
        preferred_element_type=jnp.bfloat16,
    ).reshape(B, Sq, h_per, Dh)

    K = K_ext.astype(jnp.bfloat16)
    V = V_ext.astype(jnp.bfloat16)
    scores = jnp.einsum(
        "bihd,bjhd->bhij", Q, K, preferred_element_type=jnp.float32
    ) * 0.125

    qb = (jnp.arange(Sq) // 64)[:, None]
    kb = (jnp.arange(Skv) // 64)[None, :]
    mask = (qb == kb) | (kb == 0) | ((qb + kb) % 3 == 0)
    scores = jnp.where(mask[None, None], scores, -1e9)
    smax = scores.max(axis=-1, keepdims=True)
    w = jnp.exp(scores - smax)
    row_keep = mask.any(axis=1)
    w_sum = jnp.where(
        row_keep[None, None, :, None], w.sum(axis=-1, keepdims=True), 1.0
    )
    w = jnp.where(row_keep[None, None, :, None], w / w_sum, 0.0)

    ctx = jnp.einsum(
        "bhij,bjhd->bihd", w.astype(jnp.bfloat16), V,
        preferred_element_type=jnp.bfloat16,
    ).reshape(B, Sq, cols)
    part = jnp.einsum(
        "bsc,cd->bsd", ctx, Wo_l.astype(jnp.bfloat16),
        preferred_element_type=jnp.float32,
    )

    out = _allreduce_sum(part.reshape(B * Sq, d_model))
    return out.reshape(B, Sq, d_model)


# device time: 31345 ns/iter; 3.2373x vs baseline; 1.2301x over previous
import jax
import jax.numpy as jnp
from jax import lax
from jax.experimental import pallas as pl
from jax.experimental.pallas import tpu as pltpu

N_DEV = 32
LOG_N = 5


def _allreduce_sum(partial):
    m, n = partial.shape
    rows = m // N_DEV

    def body(
        in_ref,
        out_ref,
        sendbuf,
        slots,
        ag_ref,
        red_ref,
        rs_send_sems,
        rs_recv_sems,
        ag_send_sems,
        ag_recv_sems,
    ):
        my = lax.axis_index("i")

        barrier = pltpu.get_barrier_semaphore()
        for o in range(1, N_DEV):
            pl.semaphore_signal(
                barrier,
                inc=1,
                device_id=((my + o) % N_DEV,),
                device_id_type=pl.DeviceIdType.MESH,
            )
        pl.semaphore_wait(barrier, N_DEV - 1)

        sendbuf[...] = in_ref[...].astype(jnp.bfloat16)
        rs = []
        for o in range(1, N_DEV):
            dst = (my + o) % N_DEV
            rdma = pltpu.make_async_remote_copy(
                src_ref=sendbuf.at[pl.ds(dst * rows, rows), :],
                dst_ref=slots.at[N_DEV - o],
                send_sem=rs_send_sems.at[o],
                recv_sem=rs_recv_sems.at[N_DEV - o],
                device_id=(dst,),
                device_id_type=pl.DeviceIdType.MESH,
            )
            rdma.start()
            rs.append(rdma)
        for s in range(1, N_DEV):
            pltpu.make_async_remote_copy(
                src_ref=slots.at[s],
                dst_ref=slots.at[s],
                send_sem=rs_send_sems.at[0],
                recv_sem=rs_recv_sems.at[s],
                device_id=(my,),
                device_id_type=pl.DeviceIdType.MESH,
            ).wait_recv()

        red_ref[...] = (
            in_ref[pl.ds(my * rows, rows), :]
            + slots[1:N_DEV, :, :].astype(jnp.float32).sum(axis=0)
        )
        ag_ref[pl.ds(my * rows, rows), :] = red_ref[...].astype(jnp.bfloat16)

        ag = []
        for o in range(1, N_DEV):
            dst = (my + o) % N_DEV
            rdma = pltpu.make_async_remote_copy(
                src_ref=ag_ref.at[pl.ds(my * rows, rows), :],
                dst_ref=ag_ref.at[pl.ds(my * rows, rows), :],
                send_sem=ag_send_sems.at[o],
                recv_sem=ag_recv_sems.at[N_DEV - o],
                device_id=(dst,),
                device_id_type=pl.DeviceIdType.MESH,
            )
            rdma.start()
            ag.append(rdma)
        for s in range(1, N_DEV):
            pltpu.make_async_remote_copy(
                src_ref=slots.at[s],
                dst_ref=slots.at[s],
                send_sem=ag_send_sems.at[0],
                recv_sem=ag_recv_sems.at[s],
                device_id=(my,),
                device_id_type=pl.DeviceIdType.MESH,
            ).wait_recv()

        out_ref[...] = ag_ref[...].astype(jnp.float32)

        for r in rs:
            r.wait_send()
        for r in ag:
            r.wait_send()

    return pl.pallas_call(
        body,
        out_shape=jax.ShapeDtypeStruct((m, n), partial.dtype),
        in_specs=[pl.BlockSpec(memory_space=pltpu.VMEM)],
        out_specs=pl.BlockSpec(memory_space=pltpu.VMEM),
        scratch_shapes=[
            pltpu.VMEM((m, n), jnp.bfloat16),
            pltpu.VMEM((N_DEV, rows, n), jnp.bfloat16),
            pltpu.VMEM((m, n), jnp.bfloat16),
            pltpu.VMEM((rows, n), jnp.float32),
            pltpu.SemaphoreType.DMA((N_DEV,)),
            pltpu.SemaphoreType.DMA((N_DEV,)),
            pltpu.SemaphoreType.DMA((N_DEV,)),
            pltpu.SemaphoreType.DMA((N_DEV,)),
        ],
        compiler_params=pltpu.CompilerParams(collective_id=0),
    )(partial)


def kernel(x, Wq, K_ext, V_ext, Wo):
    B, Sq, d_model = x.shape
    _, Skv, h_per, Dh = K_ext.shape
    cols = h_per * Dh

    my = lax.axis_index("i")
    Wq_l = lax.dynamic_slice(Wq, (0, my * cols), (d_model, cols))
    Wo_l = lax.dynamic_slice(Wo, (my * cols, 0), (cols, Wo.shape[1]))

    xb = x.astype(jnp.bfloat16)
    Q = jnp.einsum(
        "bsd,dc->bsc", xb, Wq_l.astype(jnp.bfloat16),
        preferred_element_type=jnp.bfloat16,
    ).reshape(B, Sq, h_per, Dh)

    K = K_ext.astype(jnp.bfloat16)
    V = V_ext.astype(jnp.bfloat16)
    scores = jnp.einsum(
        "bihd,bjhd->bhij", Q, K, preferred_element_type=jnp.float32
    ) * 0.125

    qb = (jnp.arange(Sq) // 64)[:, None]
    kb = (jnp.arange(Skv) // 64)[None, :]
    mask = (qb == kb) | (kb == 0) | ((qb + kb) % 3 == 0)
    scores = jnp.where(mask[None, None], scores, -1e9)
    smax = scores.max(axis=-1, keepdims=True)
    w = jnp.exp(scores - smax)
    row_keep = mask.any(axis=1)
    w_sum = jnp.where(
        row_keep[None, None, :, None], w.sum(axis=-1, keepdims=True), 1.0
    )
    w = jnp.where(row_keep[None, None, :, None], w / w_sum, 0.0)

    ctx = jnp.einsum(
        "bhij,bjhd->bihd", w.astype(jnp.bfloat16), V,
        preferred_element_type=jnp.bfloat16,
    ).reshape(B, Sq, cols)
    part = jnp.einsum(
        "bsc,cd->bsd", ctx, Wo_l.astype(jnp.bfloat16),
        preferred_element_type=jnp.float32,
    )

    out = _allreduce_sum(part.reshape(B * Sq, d_model))
    return out.reshape(B, Sq, d_model)
